# Initial kernel scaffold; baseline (speedup 1.0000x reference)
import jax
import jax.numpy as jnp
from jax.experimental import pallas as pl


def kernel(x, edge_index, batch, W1, b1, W2, b2, W3, b3, Wl, bl):
    raise NotImplementedError("write your pallas kernel here")


# speedup vs baseline: 24.7391x; 24.7391x over previous
import functools

import jax
import jax.numpy as jnp
from jax import lax
from jax.experimental import pallas as pl
from jax.experimental.pallas import tpu as pltpu
from jax.experimental.pallas import tpu_sc as plsc

N = 10000
E = 320000
D = 128
G = 64
C = 10

NC = 2
NS = 16
NW = NC * NS
NP = 10240
RPT = NP // NS
ET = E // NW
K = 125
NCH = ET // K
PH = 2
PCH = NCH // PH
DW = 128

BR = 1024
NB = NP // BR


@functools.cache
def _sc_deg_kernel():
    mesh = plsc.VectorSubcoreMesh(core_axis_name="c", subcore_axis_name="s")
    return pl.kernel(
        _sc_deg_body,
        out_type=jax.ShapeDtypeStruct((NC * NP, DW), jnp.float32),
        mesh=mesh,
        scratch_types=[
            pltpu.VMEM_SHARED((NP, DW), jnp.float32),
            pltpu.VMEM((NCH, K), jnp.int32),
            pltpu.VMEM((K, DW), jnp.float32),
        ],
    )


def _sc_deg(dst3, ones16, z16):
    return _sc_deg_kernel()(dst3, ones16, z16)


def _sc_deg_body(dst3_h, ones_h, z_h, out_h, acc, didx_v, ones_v):
    c = lax.axis_index("c")
    s = lax.axis_index("s")
    t = c * NS + s
    pltpu.sync_copy(z_h, acc.at[pl.ds(s * RPT, RPT)])
    pltpu.sync_copy(dst3_h.at[t], didx_v)
    pltpu.sync_copy(ones_h, ones_v)
    plsc.subcore_barrier()

    def body(j, carry):
        pltpu.sync_copy(ones_v, acc.at[didx_v.at[j]], add=True)
        return carry

    lax.fori_loop(0, NCH, body, 0)
    plsc.subcore_barrier()
    pltpu.sync_copy(acc.at[pl.ds(s * RPT, RPT)],
                    out_h.at[pl.ds(c * NP + s * RPT, RPT)])


@functools.cache
def _sc_edge_kernel():
    mesh = plsc.VectorSubcoreMesh(core_axis_name="c", subcore_axis_name="s")
    return pl.kernel(
        _sc_edge_body,
        out_type=jax.ShapeDtypeStruct((NC * NP, D), jnp.float32),
        mesh=mesh,
        scratch_types=[
            pltpu.VMEM_SHARED((NP, D), jnp.float32),
            pltpu.VMEM((PCH, K), jnp.int32),
            pltpu.VMEM((PCH, K), jnp.int32),
            pltpu.VMEM((K, D), jnp.float32),
            pltpu.VMEM((K, D), jnp.float32),
            pltpu.SemaphoreType.DMA,
            pltpu.SemaphoreType.DMA,
        ],
    )


def _sc_edge(y, src4, dst4, z128):
    return _sc_edge_kernel()(y, src4, dst4, z128)


def _sc_edge_body(y_h, src4_h, dst4_h, z_h, out_h,
                  acc, sidx_v, didx_v, rows0, rows1, sem0, sem1):
    c = lax.axis_index("c")
    s = lax.axis_index("s")
    t = c * NS + s
    pltpu.sync_copy(z_h, acc.at[pl.ds(s * RPT, RPT)])

    bufs = ((rows0, sem0), (rows1, sem1))

    for p in range(PH):
        pltpu.sync_copy(src4_h.at[t * PH + p], sidx_v)
        pltpu.sync_copy(dst4_h.at[t * PH + p], didx_v)
        if p == 0:
            plsc.subcore_barrier()
        pltpu.async_copy(y_h.at[sidx_v.at[0]], rows0, sem0)
        pltpu.async_copy(y_h.at[sidx_v.at[1]], rows1, sem1)

        def body(jj, carry):
            for b in range(2):
                rows, sem = bufs[b]
                ch = 2 * jj + b
                pltpu.make_async_copy(y_h.at[sidx_v.at[0]], rows, sem).wait()
                pltpu.sync_copy(rows, acc.at[didx_v.at[ch]], add=True)
                nxt = ch + 2

                @pl.when(nxt < PCH)
                def _prefetch():
                    pltpu.async_copy(y_h.at[sidx_v.at[nxt]], rows, sem)
            return carry

        lax.fori_loop(0, PCH // 2, body, 0)

    plsc.subcore_barrier()
    pltpu.sync_copy(acc.at[pl.ds(s * RPT, RPT)],
                    out_h.at[pl.ds(c * NP + s * RPT, RPT)])


def _k1_body(x_ref, w_ref, d0_ref, d1_ref, y_ref, dis_ref):
    dis = lax.rsqrt(d0_ref[:, 0:1] + d1_ref[:, 0:1] + 1.0)
    y_ref[...] = jnp.dot(x_ref[...], w_ref[...],
                         preferred_element_type=jnp.float32) * dis
    dis_ref[...] = jnp.broadcast_to(dis, (BR, 8))


def _tc_first(x_p, w1, degp):
    return pl.pallas_call(
        _k1_body,
        grid=(NB,),
        in_specs=[
            pl.BlockSpec((BR, D), lambda i: (i, 0)),
            pl.BlockSpec((D, D), lambda i: (0, 0)),
            pl.BlockSpec((BR, DW), lambda i: (i, 0)),
            pl.BlockSpec((BR, DW), lambda i: (i + NB, 0)),
        ],
        out_specs=[
            pl.BlockSpec((BR, D), lambda i: (i, 0)),
            pl.BlockSpec((BR, 8), lambda i: (i, 0)),
        ],
        out_shape=[
            jax.ShapeDtypeStruct((NP, D), jnp.float32),
            jax.ShapeDtypeStruct((NP, 8), jnp.float32),
        ],
    )(x_p, w1, degp, degp)


def _mid_body(a0_ref, a1_ref, y_ref, dis_ref, b_ref, w_ref, o_ref):
    dis = dis_ref[:, 0:1]
    h = dis * (a0_ref[...] + a1_ref[...] + y_ref[...]) + b_ref[...]
    h = jnp.maximum(h, 0.0)
    o_ref[...] = jnp.dot(h, w_ref[...],
                         preferred_element_type=jnp.float32) * dis


def _tc_mid(aggp, y, dis8, b, w):
    return pl.pallas_call(
        _mid_body,
        grid=(NB,),
        in_specs=[
            pl.BlockSpec((BR, D), lambda i: (i, 0)),
            pl.BlockSpec((BR, D), lambda i: (i + NB, 0)),
            pl.BlockSpec((BR, D), lambda i: (i, 0)),
            pl.BlockSpec((BR, 8), lambda i: (i, 0)),
            pl.BlockSpec((1, D), lambda i: (0, 0)),
            pl.BlockSpec((D, D), lambda i: (0, 0)),
        ],
        out_specs=pl.BlockSpec((BR, D), lambda i: (i, 0)),
        out_shape=jax.ShapeDtypeStruct((NP, D), jnp.float32),
    )(aggp, aggp, y, dis8, b, w)


def _final_body(a0_ref, a1_ref, y_ref, dis_ref, b_ref, batch_ref,
                wl_ref, bl_ref, o_ref, s_acc, c_acc):
    i = pl.program_id(0)
    dis = dis_ref[:, 0:1]
    h = dis * (a0_ref[...] + a1_ref[...] + y_ref[...]) + b_ref[...]
    h = jnp.maximum(h, 0.0)
    gids = lax.broadcasted_iota(jnp.int32, (BR, G), 1)
    onehot = (batch_ref[...] == gids).astype(jnp.float32)
    dn = (((0,), (0,)), ((), ()))
    ps = lax.dot_general(onehot, h, dn, preferred_element_type=jnp.float32)
    pc = lax.dot_general(onehot, jnp.ones((BR, D), jnp.float32), dn,
                         preferred_element_type=jnp.float32)

    @pl.when(i == 0)
    def _init():
        s_acc[...] = ps
        c_acc[...] = pc

    @pl.when(i > 0)
    def _accum():
        s_acc[...] += ps
        c_acc[...] += pc

    @pl.when(i == NB - 1)
    def _head():
        pooled = s_acc[...] / jnp.maximum(c_acc[...], 1.0)
        o_ref[...] = jnp.dot(pooled, wl_ref[...],
                             preferred_element_type=jnp.float32) + bl_ref[...]


def _tc_final(aggp, y, dis8, b, batch_p, wl, bl):
    return pl.pallas_call(
        _final_body,
        grid=(NB,),
        in_specs=[
            pl.BlockSpec((BR, D), lambda i: (i, 0)),
            pl.BlockSpec((BR, D), lambda i: (i + NB, 0)),
            pl.BlockSpec((BR, D), lambda i: (i, 0)),
            pl.BlockSpec((BR, 8), lambda i: (i, 0)),
            pl.BlockSpec((1, D), lambda i: (0, 0)),
            pl.BlockSpec((BR, 1), lambda i: (i, 0)),
            pl.BlockSpec((D, C), lambda i: (0, 0)),
            pl.BlockSpec((1, C), lambda i: (0, 0)),
        ],
        out_specs=pl.BlockSpec((G, C), lambda i: (0, 0)),
        out_shape=jax.ShapeDtypeStruct((G, C), jnp.float32),
        scratch_shapes=[
            pltpu.VMEM((G, D), jnp.float32),
            pltpu.VMEM((G, D), jnp.float32),
        ],
    )(aggp, aggp, y, dis8, b, batch_p, wl, bl)


def kernel(x, edge_index, batch, W1, b1, W2, b2, W3, b3, Wl, bl):
    src3 = edge_index[0].astype(jnp.int32).reshape(NW, NCH, K)
    dst3 = edge_index[1].astype(jnp.int32).reshape(NW, NCH, K)
    src4 = src3.reshape(NW * PH, PCH, K)
    dst4 = dst3.reshape(NW * PH, PCH, K)
    x_p = jnp.pad(x, ((0, NP - N), (0, 0)))
    batch_p = jnp.pad(batch.astype(jnp.int32), (0, NP - N),
                      constant_values=G).reshape(NP, 1)
    z128 = jnp.zeros((RPT, D), jnp.float32)
    ones128 = jnp.ones((K, DW), jnp.float32)

    degp = _sc_deg(dst3, ones128, z128)
    y1, dis8 = _tc_first(x_p, W1, degp)
    a1 = _sc_edge(y1, src4, dst4, z128)
    y2 = _tc_mid(a1, y1, dis8, b1.reshape(1, D), W2)
    a2 = _sc_edge(y2, src4, dst4, z128)
    y3 = _tc_mid(a2, y2, dis8, b2.reshape(1, D), W3)
    a3 = _sc_edge(y3, src4, dst4, z128)
    return _tc_final(a3, y3, dis8, b3.reshape(1, D), batch_p, Wl,
                     bl.reshape(1, C))


# speedup vs baseline: 24.8054x; 1.0027x over previous
import functools

import jax
import jax.numpy as jnp
from jax import lax
from jax.experimental import pallas as pl
from jax.experimental.pallas import tpu as pltpu
from jax.experimental.pallas import tpu_sc as plsc

N = 10000
E = 320000
D = 128
G = 64
C = 10

NC = 2
NS = 16
NW = NC * NS
NP = 10240
RPT = NP // NS
ET = E // NW
KD = 125
NCHD = ET // KD
K = 80
NCH = ET // K
PH = 5
PCH = NCH // PH
NBUF = 3
DW = 128

BR = 1024
NB = NP // BR


@functools.cache
def _sc_deg_kernel():
    mesh = plsc.VectorSubcoreMesh(core_axis_name="c", subcore_axis_name="s")
    return pl.kernel(
        _sc_deg_body,
        out_type=jax.ShapeDtypeStruct((NC * NP, DW), jnp.float32),
        mesh=mesh,
        scratch_types=[
            pltpu.VMEM_SHARED((NP, DW), jnp.float32),
            pltpu.VMEM((NCHD, KD), jnp.int32),
            pltpu.VMEM((KD, DW), jnp.float32),
            pltpu.SemaphoreType.DMA,
            pltpu.SemaphoreType.DMA,
            pltpu.SemaphoreType.DMA,
            pltpu.SemaphoreType.DMA,
        ],
    )


def _sc_deg(dst3, ones_f, z_f):
    return _sc_deg_kernel()(dst3, ones_f, z_f)


def _sc_deg_body(dst3_h, ones_h, z_h, out_h, acc, didx_v, ones_v,
                 d0, d1, d2, d3):
    c = lax.axis_index("c")
    s = lax.axis_index("s")
    t = c * NS + s
    pltpu.sync_copy(z_h, acc.at[pl.ds(s * RPT, RPT)])
    pltpu.sync_copy(dst3_h.at[t], didx_v)
    pltpu.sync_copy(ones_h, ones_v)
    plsc.subcore_barrier()

    sems = (d0, d1, d2, d3)

    def body(j, carry):
        for b in range(4):
            ch = 4 * j + b

            @pl.when(ch >= 4)
            def _drain():
                pltpu.make_async_copy(ones_v, acc.at[didx_v.at[0]],
                                      sems[b]).wait()

            pltpu.async_copy(ones_v, acc.at[didx_v.at[ch]], sems[b],
                             add=True)
        return carry

    lax.fori_loop(0, NCHD // 4, body, 0)
    for b in range(4):
        pltpu.make_async_copy(ones_v, acc.at[didx_v.at[0]], sems[b]).wait()
    plsc.subcore_barrier()
    pltpu.sync_copy(acc.at[pl.ds(s * RPT, RPT)],
                    out_h.at[pl.ds(c * NP + s * RPT, RPT)])


@functools.cache
def _sc_edge_kernel():
    mesh = plsc.VectorSubcoreMesh(core_axis_name="c", subcore_axis_name="s")
    return pl.kernel(
        _sc_edge_body,
        out_type=jax.ShapeDtypeStruct((NC * NP, D), jnp.float32),
        mesh=mesh,
        scratch_types=[
            pltpu.VMEM_SHARED((NP, D), jnp.float32),
            pltpu.VMEM((PCH, K), jnp.int32),
            pltpu.VMEM((PCH, K), jnp.int32),
            [pltpu.VMEM((K, D), jnp.float32) for _ in range(NBUF)],
            [pltpu.SemaphoreType.DMA for _ in range(NBUF)],
            [pltpu.SemaphoreType.DMA for _ in range(NBUF)],
        ],
    )


def _sc_edge(y, src4, dst4, z128):
    return _sc_edge_kernel()(y, src4, dst4, z128)


def _sc_edge_body(y_h, src4_h, dst4_h, z_h, out_h,
                  acc, sidx_v, didx_v, rows, gsem, ssem):
    c = lax.axis_index("c")
    s = lax.axis_index("s")
    t = c * NS + s
    pltpu.sync_copy(z_h, acc.at[pl.ds(s * RPT, RPT)])

    def gwait(b):
        pltpu.make_async_copy(y_h.at[sidx_v.at[0]], rows[b], gsem[b]).wait()

    def swait(b):
        pltpu.make_async_copy(rows[b], acc.at[didx_v.at[0]], ssem[b]).wait()

    for p in range(PH):
        pltpu.sync_copy(src4_h.at[t * PH + p], sidx_v)
        pltpu.sync_copy(dst4_h.at[t * PH + p], didx_v)
        if p == 0:
            plsc.subcore_barrier()
        for b in range(NBUF):
            pltpu.async_copy(y_h.at[sidx_v.at[b]], rows[b], gsem[b])

        def body(jj, carry):
            for b in range(NBUF):
                ch = NBUF * jj + b

                @pl.when(ch < PCH)
                def _chunk():
                    gwait(b)
                    pltpu.async_copy(rows[b], acc.at[didx_v.at[ch]],
                                     ssem[b], add=True)
                    nxt = ch + NBUF

                    @pl.when(nxt < PCH)
                    def _prefetch():
                        swait(b)
                        pltpu.async_copy(y_h.at[sidx_v.at[nxt]], rows[b],
                                         gsem[b])
            return carry

        lax.fori_loop(0, (PCH + NBUF - 1) // NBUF, body, 0)
        for b in range(NBUF):
            swait(b)

    plsc.subcore_barrier()
    pltpu.sync_copy(acc.at[pl.ds(s * RPT, RPT)],
                    out_h.at[pl.ds(c * NP + s * RPT, RPT)])


def _k1_body(x_ref, w_ref, d0_ref, d1_ref, y_ref, dis_ref):
    dis = lax.rsqrt(d0_ref[:, 0:1] + d1_ref[:, 0:1] + 1.0)
    y_ref[...] = jnp.dot(x_ref[...], w_ref[...],
                         preferred_element_type=jnp.float32) * dis
    dis_ref[...] = jnp.broadcast_to(dis, (BR, 8))


def _tc_first(x_p, w1, degp):
    return pl.pallas_call(
        _k1_body,
        grid=(NB,),
        in_specs=[
            pl.BlockSpec((BR, D), lambda i: (i, 0)),
            pl.BlockSpec((D, D), lambda i: (0, 0)),
            pl.BlockSpec((BR, DW), lambda i: (i, 0)),
            pl.BlockSpec((BR, DW), lambda i: (i + NB, 0)),
        ],
        out_specs=[
            pl.BlockSpec((BR, D), lambda i: (i, 0)),
            pl.BlockSpec((BR, 8), lambda i: (i, 0)),
        ],
        out_shape=[
            jax.ShapeDtypeStruct((NP, D), jnp.float32),
            jax.ShapeDtypeStruct((NP, 8), jnp.float32),
        ],
    )(x_p, w1, degp, degp)


def _mid_body(a0_ref, a1_ref, y_ref, dis_ref, b_ref, w_ref, o_ref):
    dis = dis_ref[:, 0:1]
    h = dis * (a0_ref[...] + a1_ref[...] + y_ref[...]) + b_ref[...]
    h = jnp.maximum(h, 0.0)
    o_ref[...] = jnp.dot(h, w_ref[...],
                         preferred_element_type=jnp.float32) * dis


def _tc_mid(aggp, y, dis8, b, w):
    return pl.pallas_call(
        _mid_body,
        grid=(NB,),
        in_specs=[
            pl.BlockSpec((BR, D), lambda i: (i, 0)),
            pl.BlockSpec((BR, D), lambda i: (i + NB, 0)),
            pl.BlockSpec((BR, D), lambda i: (i, 0)),
            pl.BlockSpec((BR, 8), lambda i: (i, 0)),
            pl.BlockSpec((1, D), lambda i: (0, 0)),
            pl.BlockSpec((D, D), lambda i: (0, 0)),
        ],
        out_specs=pl.BlockSpec((BR, D), lambda i: (i, 0)),
        out_shape=jax.ShapeDtypeStruct((NP, D), jnp.float32),
    )(aggp, aggp, y, dis8, b, w)


def _final_body(a0_ref, a1_ref, y_ref, dis_ref, b_ref, batch_ref,
                wl_ref, bl_ref, o_ref, s_acc, c_acc):
    i = pl.program_id(0)
    dis = dis_ref[:, 0:1]
    h = dis * (a0_ref[...] + a1_ref[...] + y_ref[...]) + b_ref[...]
    h = jnp.maximum(h, 0.0)
    gids = lax.broadcasted_iota(jnp.int32, (BR, G), 1)
    onehot = (batch_ref[...] == gids).astype(jnp.float32)
    dn = (((0,), (0,)), ((), ()))
    ps = lax.dot_general(onehot, h, dn, preferred_element_type=jnp.float32)
    pc = lax.dot_general(onehot, jnp.ones((BR, D), jnp.float32), dn,
                         preferred_element_type=jnp.float32)

    @pl.when(i == 0)
    def _init():
        s_acc[...] = ps
        c_acc[...] = pc

    @pl.when(i > 0)
    def _accum():
        s_acc[...] += ps
        c_acc[...] += pc

    @pl.when(i == NB - 1)
    def _head():
        pooled = s_acc[...] / jnp.maximum(c_acc[...], 1.0)
        o_ref[...] = jnp.dot(pooled, wl_ref[...],
                             preferred_element_type=jnp.float32) + bl_ref[...]


def _tc_final(aggp, y, dis8, b, batch_p, wl, bl):
    return pl.pallas_call(
        _final_body,
        grid=(NB,),
        in_specs=[
            pl.BlockSpec((BR, D), lambda i: (i, 0)),
            pl.BlockSpec((BR, D), lambda i: (i + NB, 0)),
            pl.BlockSpec((BR, D), lambda i: (i, 0)),
            pl.BlockSpec((BR, 8), lambda i: (i, 0)),
            pl.BlockSpec((1, D), lambda i: (0, 0)),
            pl.BlockSpec((BR, 1), lambda i: (i, 0)),
            pl.BlockSpec((D, C), lambda i: (0, 0)),
            pl.BlockSpec((1, C), lambda i: (0, 0)),
        ],
        out_specs=pl.BlockSpec((G, C), lambda i: (0, 0)),
        out_shape=jax.ShapeDtypeStruct((G, C), jnp.float32),
        scratch_shapes=[
            pltpu.VMEM((G, D), jnp.float32),
            pltpu.VMEM((G, D), jnp.float32),
        ],
    )(aggp, aggp, y, dis8, b, batch_p, wl, bl)


def kernel(x, edge_index, batch, W1, b1, W2, b2, W3, b3, Wl, bl):
    src4 = edge_index[0].astype(jnp.int32).reshape(NW * PH, PCH, K)
    dst4 = edge_index[1].astype(jnp.int32).reshape(NW * PH, PCH, K)
    dst3 = edge_index[1].astype(jnp.int32).reshape(NW, NCHD, KD)
    x_p = jnp.pad(x, ((0, NP - N), (0, 0)))
    batch_p = jnp.pad(batch.astype(jnp.int32), (0, NP - N),
                      constant_values=G).reshape(NP, 1)
    z128 = jnp.zeros((RPT, D), jnp.float32)
    ones_f = jnp.ones((KD, DW), jnp.float32)

    degp = _sc_deg(dst3, ones_f, z128)
    y1, dis8 = _tc_first(x_p, W1, degp)
    a1 = _sc_edge(y1, src4, dst4, z128)
    y2 = _tc_mid(a1, y1, dis8, b1.reshape(1, D), W2)
    a2 = _sc_edge(y2, src4, dst4, z128)
    y3 = _tc_mid(a2, y2, dis8, b2.reshape(1, D), W3)
    a3 = _sc_edge(y3, src4, dst4, z128)
    return _tc_final(a3, y3, dis8, b3.reshape(1, D), batch_p, Wl,
                     bl.reshape(1, C))


# speedup vs baseline: 25.5157x; 1.0286x over previous
import functools

import jax
import jax.numpy as jnp
from jax import lax
from jax.experimental import pallas as pl
from jax.experimental.pallas import tpu as pltpu
from jax.experimental.pallas import tpu_sc as plsc

N = 10000
E = 320000
D = 128
G = 64
C = 10

NC = 2
NS = 16
NW = NC * NS
NP = 10240
RPT = NP // NS
ET = E // NW
K = 80
NCH = ET // K
PH = 5
PCH = NCH // PH
NBUF = 3
DW = 128

BR = 1024
NB = NP // BR


@functools.cache
def _sc_deg_kernel():
    mesh = plsc.VectorSubcoreMesh(core_axis_name="c", subcore_axis_name="s")
    return pl.kernel(
        _sc_deg_body,
        out_type=jax.ShapeDtypeStruct((NC * NP, DW), jnp.float32),
        mesh=mesh,
        scratch_types=[
            pltpu.VMEM_SHARED((NP, DW), jnp.float32),
            pltpu.VMEM((NCH, K), jnp.int32),
            pltpu.VMEM((K, DW), jnp.float32),
            [pltpu.SemaphoreType.DMA for _ in range(4)],
        ],
    )


def _sc_deg(ef2, ones_f, z_f):
    return _sc_deg_kernel()(ef2, ones_f, z_f)


def _sc_deg_body(ef2_h, ones_h, z_h, out_h, acc, didx_v, ones_v, sems):
    c = lax.axis_index("c")
    s = lax.axis_index("s")
    t = c * NS + s
    pltpu.sync_copy(z_h, acc.at[pl.ds(s * RPT, RPT)])
    pltpu.sync_copy(ef2_h.at[NW + t], didx_v)
    pltpu.sync_copy(ones_h, ones_v)
    plsc.subcore_barrier()

    def body(j, carry):
        for b in range(4):
            ch = 4 * j + b

            @pl.when(jnp.logical_and(ch >= 4, ch < NCH))
            def _drain():
                pltpu.make_async_copy(ones_v, acc.at[didx_v.at[0]],
                                      sems[b]).wait()

            @pl.when(ch < NCH)
            def _fire():
                pltpu.async_copy(ones_v, acc.at[didx_v.at[ch]], sems[b],
                                 add=True)
        return carry

    lax.fori_loop(0, (NCH + 3) // 4, body, 0)
    for b in range(4):
        pltpu.make_async_copy(ones_v, acc.at[didx_v.at[0]], sems[b]).wait()
    plsc.subcore_barrier()
    pltpu.sync_copy(acc.at[pl.ds(s * RPT, RPT)],
                    out_h.at[pl.ds(c * NP + s * RPT, RPT)])


@functools.cache
def _sc_edge_kernel():
    mesh = plsc.VectorSubcoreMesh(core_axis_name="c", subcore_axis_name="s")
    return pl.kernel(
        _sc_edge_body,
        out_type=jax.ShapeDtypeStruct((NC * NP, D), jnp.float32),
        mesh=mesh,
        scratch_types=[
            pltpu.VMEM_SHARED((NP, D), jnp.float32),
            pltpu.VMEM((PCH, K), jnp.int32),
            pltpu.VMEM((PCH, K), jnp.int32),
            [pltpu.VMEM((K, D), jnp.float32) for _ in range(NBUF)],
            [pltpu.SemaphoreType.DMA for _ in range(NBUF)],
            [pltpu.SemaphoreType.DMA for _ in range(NBUF)],
        ],
    )


def _sc_edge(y, ef4, z128):
    return _sc_edge_kernel()(y, ef4, z128)


def _sc_edge_body(y_h, ef4_h, z_h, out_h,
                  acc, sidx_v, didx_v, rows, gsem, ssem):
    c = lax.axis_index("c")
    s = lax.axis_index("s")
    t = c * NS + s
    pltpu.sync_copy(z_h, acc.at[pl.ds(s * RPT, RPT)])

    def gwait(b):
        pltpu.make_async_copy(y_h.at[sidx_v.at[0]], rows[b], gsem[b]).wait()

    def swait(b):
        pltpu.make_async_copy(rows[b], acc.at[didx_v.at[0]], ssem[b]).wait()

    for p in range(PH):
        pltpu.sync_copy(ef4_h.at[t * PH + p], sidx_v)
        pltpu.sync_copy(ef4_h.at[NW * PH + t * PH + p], didx_v)
        if p == 0:
            plsc.subcore_barrier()
        for b in range(NBUF):
            pltpu.async_copy(y_h.at[sidx_v.at[b]], rows[b], gsem[b])

        def body(jj, carry):
            for b in range(NBUF):
                ch = NBUF * jj + b

                @pl.when(ch < PCH)
                def _chunk():
                    gwait(b)
                    pltpu.async_copy(rows[b], acc.at[didx_v.at[ch]],
                                     ssem[b], add=True)
                    nxt = ch + NBUF

                    @pl.when(nxt < PCH)
                    def _prefetch():
                        swait(b)
                        pltpu.async_copy(y_h.at[sidx_v.at[nxt]], rows[b],
                                         gsem[b])
            return carry

        lax.fori_loop(0, (PCH + NBUF - 1) // NBUF, body, 0)
        for b in range(NBUF):
            swait(b)

    plsc.subcore_barrier()
    pltpu.sync_copy(acc.at[pl.ds(s * RPT, RPT)],
                    out_h.at[pl.ds(c * NP + s * RPT, RPT)])


def _k1_body(x_ref, w_ref, d0_ref, d1_ref, y_ref, dis_ref):
    dis = lax.rsqrt(d0_ref[:, 0:1] + d1_ref[:, 0:1] + 1.0)
    y_ref[...] = jnp.dot(x_ref[...], w_ref[...],
                         preferred_element_type=jnp.float32) * dis
    dis_ref[...] = jnp.broadcast_to(dis, (BR, 8))


def _tc_first(x, w1, degp):
    return pl.pallas_call(
        _k1_body,
        grid=(NB,),
        in_specs=[
            pl.BlockSpec((BR, D), lambda i: (i, 0)),
            pl.BlockSpec((D, D), lambda i: (0, 0)),
            pl.BlockSpec((BR, DW), lambda i: (i, 0)),
            pl.BlockSpec((BR, DW), lambda i: (i + NB, 0)),
        ],
        out_specs=[
            pl.BlockSpec((BR, D), lambda i: (i, 0)),
            pl.BlockSpec((BR, 8), lambda i: (i, 0)),
        ],
        out_shape=[
            jax.ShapeDtypeStruct((NP, D), jnp.float32),
            jax.ShapeDtypeStruct((NP, 8), jnp.float32),
        ],
    )(x, w1, degp, degp)


def _mid_body(a0_ref, a1_ref, y_ref, dis_ref, b_ref, w_ref, o_ref):
    dis = dis_ref[:, 0:1]
    h = dis * (a0_ref[...] + a1_ref[...] + y_ref[...]) + b_ref[...]
    h = jnp.maximum(h, 0.0)
    o_ref[...] = jnp.dot(h, w_ref[...],
                         preferred_element_type=jnp.float32) * dis


def _tc_mid(aggp, y, dis8, b, w):
    return pl.pallas_call(
        _mid_body,
        grid=(NB,),
        in_specs=[
            pl.BlockSpec((BR, D), lambda i: (i, 0)),
            pl.BlockSpec((BR, D), lambda i: (i + NB, 0)),
            pl.BlockSpec((BR, D), lambda i: (i, 0)),
            pl.BlockSpec((BR, 8), lambda i: (i, 0)),
            pl.BlockSpec((1, D), lambda i: (0, 0)),
            pl.BlockSpec((D, D), lambda i: (0, 0)),
        ],
        out_specs=pl.BlockSpec((BR, D), lambda i: (i, 0)),
        out_shape=jax.ShapeDtypeStruct((NP, D), jnp.float32),
    )(aggp, aggp, y, dis8, b, w)


def _final_body(a0_ref, a1_ref, y_ref, dis_ref, b_ref, batch_ref,
                wl_ref, bl_ref, o_ref, s_acc, c_acc):
    i = pl.program_id(0)
    dis = dis_ref[:, 0:1]
    h = dis * (a0_ref[...] + a1_ref[...] + y_ref[...]) + b_ref[...]
    h = jnp.maximum(h, 0.0)
    gids = lax.broadcasted_iota(jnp.int32, (BR, G), 1)
    rowid = i * BR + lax.broadcasted_iota(jnp.int32, (BR, 1), 0)
    valid = rowid < N
    h = jnp.where(valid, h, 0.0)
    onehot = jnp.where(valid, (batch_ref[...] == gids).astype(jnp.float32),
                       0.0)
    dn = (((0,), (0,)), ((), ()))
    ps = lax.dot_general(onehot, h, dn, preferred_element_type=jnp.float32)
    pc = lax.dot_general(onehot, jnp.ones((BR, D), jnp.float32), dn,
                         preferred_element_type=jnp.float32)

    @pl.when(i == 0)
    def _init():
        s_acc[...] = ps
        c_acc[...] = pc

    @pl.when(i > 0)
    def _accum():
        s_acc[...] += ps
        c_acc[...] += pc

    @pl.when(i == NB - 1)
    def _head():
        pooled = s_acc[...] / jnp.maximum(c_acc[...], 1.0)
        o_ref[...] = jnp.dot(pooled, wl_ref[...],
                             preferred_element_type=jnp.float32) + bl_ref[...]


def _tc_final(aggp, y, dis8, b, batch_p, wl, bl):
    return pl.pallas_call(
        _final_body,
        grid=(NB,),
        in_specs=[
            pl.BlockSpec((BR, D), lambda i: (i, 0)),
            pl.BlockSpec((BR, D), lambda i: (i + NB, 0)),
            pl.BlockSpec((BR, D), lambda i: (i, 0)),
            pl.BlockSpec((BR, 8), lambda i: (i, 0)),
            pl.BlockSpec((1, D), lambda i: (0, 0)),
            pl.BlockSpec((BR, 1), lambda i: (i, 0)),
            pl.BlockSpec((D, C), lambda i: (0, 0)),
            pl.BlockSpec((1, C), lambda i: (0, 0)),
        ],
        out_specs=pl.BlockSpec((G, C), lambda i: (0, 0)),
        out_shape=jax.ShapeDtypeStruct((G, C), jnp.float32),
        scratch_shapes=[
            pltpu.VMEM((G, D), jnp.float32),
            pltpu.VMEM((G, D), jnp.float32),
        ],
    )(aggp, aggp, y, dis8, b, batch_p, wl, bl)


def kernel(x, edge_index, batch, W1, b1, W2, b2, W3, b3, Wl, bl):
    ei = edge_index.astype(jnp.int32)
    ef4 = ei.reshape(2 * NW * PH, PCH, K)
    ef2 = ei.reshape(2 * NW, NCH, K)
    batch_c = batch.astype(jnp.int32).reshape(N, 1)
    z128 = jnp.zeros((RPT, D), jnp.float32)
    ones_f = jnp.ones((K, DW), jnp.float32)

    degp = _sc_deg(ef2, ones_f, z128)
    y1, dis8 = _tc_first(x, W1, degp)
    a1 = _sc_edge(y1, ef4, z128)
    y2 = _tc_mid(a1, y1, dis8, b1.reshape(1, D), W2)
    a2 = _sc_edge(y2, ef4, z128)
    y3 = _tc_mid(a2, y2, dis8, b2.reshape(1, D), W3)
    a3 = _sc_edge(y3, ef4, z128)
    return _tc_final(a3, y3, dis8, b3.reshape(1, D), batch_c, Wl,
                     bl.reshape(1, C))
